# baseline (speedup 1.0000x reference)
import jax
import jax.numpy as jnp
import numpy as np
from jax.experimental import pallas as pl
from jax.experimental.pallas import tpu as pltpu

_B, _P, _A, _E, _C = 128, 8, 12, 512, 2
_EC = _E * _C
_AC = _A * _C
_S = _A + 1
_BB = 8
_RB = _BB * _P * _S
_G = _BB * _P
_DELTA = 1e-20


def _body(back_ref, fwd_ref, wsel_ref, bsel_ref, dm_ref, s64_ref, p24_ref,
          t64_ref, perm_ref, pif_ref, rew_ref, iflow_ref, out_ref):
    wsel = wsel_ref[...]
    bsel = bsel_ref[...]

    y = jnp.dot(back_ref[...], wsel,
                preferred_element_type=jnp.float32) + bsel
    sp = jax.nn.softplus(y)
    masked = sp * dm_ref[...]
    g1 = jnp.dot(s64_ref[...], masked,
                 preferred_element_type=jnp.float32)
    f_in = jnp.dot(g1, p24_ref[...],
                   preferred_element_type=jnp.float32)

    fwd = fwd_ref[...].reshape(_G, _EC)
    yf = jnp.dot(fwd, wsel, preferred_element_type=jnp.float32) + bsel
    f_out = jnp.dot(jax.nn.softplus(yf), p24_ref[...],
                    preferred_element_type=jnp.float32)

    rew = rew_ref[...]
    pif = pif_ref[...]
    f_init = pif * jnp.exp(iflow_ref[...])

    logterm = jnp.log(_DELTA + f_out) - jnp.log(_DELTA + f_out + rew)
    p_out = jnp.dot(t64_ref[...], logterm,
                    preferred_element_type=jnp.float32)

    cat = jnp.concatenate([f_in, f_out, rew, f_init, p_out, rew], axis=1)
    out_ref[...] = jnp.dot(cat, perm_ref[...],
                           preferred_element_type=jnp.float32)


def kernel(forward_edges, backward_edges, path_init_flow, paths_reward, W, b,
           initial_flow):
    back2 = backward_edges.reshape(_B * _P * _S, _EC)
    fwd2 = forward_edges.reshape(_B, _P, _S * _EC)

    wt = jnp.transpose(W, (1, 0, 2))
    eye = jnp.eye(_C, dtype=W.dtype)
    wsel = (wt[:, :, :, None] * eye[None, :, None, :]).reshape(_EC, _AC)
    bsel = jnp.transpose(b).reshape(1, _AC)
    iflow = initial_flow.reshape(1, _C)

    r = np.arange(_RB)
    j = np.arange(_AC)
    s = r % _S
    dm = ((s[:, None] >= 1) & (j[None, :] // _C == s[:, None] - 1)
          ).astype(np.float32)
    s64 = (r[None, :] // _S == np.arange(_G)[:, None]).astype(np.float32)
    p24 = (j[:, None] % _C == np.arange(_C)[None, :]).astype(np.float32)
    g = np.arange(_G)
    t64 = ((g[:, None] // _P == g[None, :] // _P)
           & (g[None, :] % _P < g[:, None] % _P)).astype(np.float32)
    kk = np.arange(12)
    perm = ((kk[:, None] % _C) * 6 + kk[:, None] // _C == kk[None, :]
            ).astype(np.float32)

    grid = (_B // _BB,)
    const = lambda arr: (jnp.asarray(arr),)
    out = pl.pallas_call(
        _body,
        grid=grid,
        in_specs=[
            pl.BlockSpec((_RB, _EC), lambda i: (i, 0)),
            pl.BlockSpec((_BB, _P, _EC), lambda i: (i, 0, 0)),
            pl.BlockSpec((_EC, _AC), lambda i: (0, 0)),
            pl.BlockSpec((1, _AC), lambda i: (0, 0)),
            pl.BlockSpec((_RB, _AC), lambda i: (0, 0)),
            pl.BlockSpec((_G, _RB), lambda i: (0, 0)),
            pl.BlockSpec((_AC, _C), lambda i: (0, 0)),
            pl.BlockSpec((_G, _G), lambda i: (0, 0)),
            pl.BlockSpec((12, 12), lambda i: (0, 0)),
            pl.BlockSpec((_G, _C), lambda i: (i, 0)),
            pl.BlockSpec((_G, _C), lambda i: (i, 0)),
            pl.BlockSpec((1, _C), lambda i: (0, 0)),
        ],
        out_specs=pl.BlockSpec((_G, 12), lambda i: (i, 0)),
        out_shape=jax.ShapeDtypeStruct((_B * _P, 12), jnp.float32),
        compiler_params=pltpu.CompilerParams(
            dimension_semantics=("arbitrary",)),
    )(back2, fwd2, wsel, bsel, jnp.asarray(dm), jnp.asarray(s64),
      jnp.asarray(p24), jnp.asarray(t64), jnp.asarray(perm),
      path_init_flow.reshape(_B * _P, _C), paths_reward.reshape(_B * _P, _C),
      iflow)
    return out.reshape(_B, _P, _C, 6)


# speedup vs baseline: 12.6400x; 12.6400x over previous
import jax
import jax.numpy as jnp
import numpy as np
from jax.experimental import pallas as pl
from jax.experimental.pallas import tpu as pltpu

_B, _P, _A, _E, _C = 128, 8, 12, 512, 2
_EC = _E * _C
_AC = _A * _C
_S = _A + 1
_BB = 8
_RB = _BB * _P * _S
_G = _BB * _P
_DELTA = 1e-20


def _body(back_ref, fwd_ref, wsel_ref, bsel_ref, dm_ref, s64_ref, p24_ref,
          t64_ref, perm_ref, pif_ref, rew_ref, iflow_ref, out_ref):
    wsel = wsel_ref[...]
    bsel = bsel_ref[...]

    back = back_ref[...].reshape(_RB, _EC)
    y = jnp.dot(back, wsel,
                preferred_element_type=jnp.float32) + bsel
    sp = jax.nn.softplus(y)
    masked = sp * dm_ref[...]
    g1 = jnp.dot(s64_ref[...], masked,
                 preferred_element_type=jnp.float32)
    f_in = jnp.dot(g1, p24_ref[...],
                   preferred_element_type=jnp.float32)

    fwd = fwd_ref[...].reshape(_G, _EC)
    yf = jnp.dot(fwd, wsel, preferred_element_type=jnp.float32) + bsel
    f_out = jnp.dot(jax.nn.softplus(yf), p24_ref[...],
                    preferred_element_type=jnp.float32)

    rew = rew_ref[...]
    pif = pif_ref[...]
    f_init = pif * jnp.exp(iflow_ref[...])

    logterm = jnp.log(_DELTA + f_out) - jnp.log(_DELTA + f_out + rew)
    p_out = jnp.dot(t64_ref[...], logterm,
                    preferred_element_type=jnp.float32)

    cat = jnp.concatenate([f_in, f_out, rew, f_init, p_out, rew], axis=1)
    out_ref[...] = jnp.dot(cat, perm_ref[...],
                           preferred_element_type=jnp.float32)


def kernel(forward_edges, backward_edges, path_init_flow, paths_reward, W, b,
           initial_flow):
    back3 = (backward_edges.reshape(_B, _P, _S, 4, 128, _C)
             .swapaxes(4, 5).reshape(_B * _P * _S, 8, 128))
    fwd4 = (forward_edges.reshape(_B, _P, _S, 4, 128, _C)
            .swapaxes(4, 5).reshape(_B, _P, _S * 8, 128))

    wr = jnp.transpose(W.reshape(_C, 4, 128, _A), (1, 0, 2, 3))
    eye = jnp.eye(_C, dtype=W.dtype)
    wsel = (wr[:, :, :, :, None] * eye[None, :, None, None, :]
            ).reshape(_EC, _AC)
    bsel = jnp.transpose(b).reshape(1, _AC)
    iflow = initial_flow.reshape(1, _C)

    r = np.arange(_RB)
    j = np.arange(_AC)
    s = r % _S
    dm = ((s[:, None] >= 1) & (j[None, :] // _C == s[:, None] - 1)
          ).astype(np.float32)
    s64 = (r[None, :] // _S == np.arange(_G)[:, None]).astype(np.float32)
    p24 = (j[:, None] % _C == np.arange(_C)[None, :]).astype(np.float32)
    g = np.arange(_G)
    t64 = ((g[:, None] // _P == g[None, :] // _P)
           & (g[None, :] % _P < g[:, None] % _P)).astype(np.float32)
    kk = np.arange(12)
    perm = ((kk[:, None] % _C) * 6 + kk[:, None] // _C == kk[None, :]
            ).astype(np.float32)

    grid = (_B // _BB,)
    out = pl.pallas_call(
        _body,
        grid=grid,
        in_specs=[
            pl.BlockSpec((_RB, 8, 128), lambda i: (i, 0, 0)),
            pl.BlockSpec((_BB, _P, 8, 128), lambda i: (i, 0, 0, 0)),
            pl.BlockSpec((_EC, _AC), lambda i: (0, 0)),
            pl.BlockSpec((1, _AC), lambda i: (0, 0)),
            pl.BlockSpec((_RB, _AC), lambda i: (0, 0)),
            pl.BlockSpec((_G, _RB), lambda i: (0, 0)),
            pl.BlockSpec((_AC, _C), lambda i: (0, 0)),
            pl.BlockSpec((_G, _G), lambda i: (0, 0)),
            pl.BlockSpec((12, 12), lambda i: (0, 0)),
            pl.BlockSpec((_G, _C), lambda i: (i, 0)),
            pl.BlockSpec((_G, _C), lambda i: (i, 0)),
            pl.BlockSpec((1, _C), lambda i: (0, 0)),
        ],
        out_specs=pl.BlockSpec((_G, 12), lambda i: (i, 0)),
        out_shape=jax.ShapeDtypeStruct((_B * _P, 12), jnp.float32),
        compiler_params=pltpu.CompilerParams(
            dimension_semantics=("arbitrary",)),
    )(back3, fwd4, wsel, bsel, jnp.asarray(dm), jnp.asarray(s64),
      jnp.asarray(p24), jnp.asarray(t64), jnp.asarray(perm),
      path_init_flow.reshape(_B * _P, _C), paths_reward.reshape(_B * _P, _C),
      iflow)
    return out.reshape(_B, _P, _C, 6)
